# Initial kernel scaffold; baseline (speedup 1.0000x reference)
import jax
import jax.numpy as jnp
from jax.experimental import pallas as pl


def kernel(x, edge_index, W):
    raise NotImplementedError("write your pallas kernel here")


# speedup vs baseline: 19.1335x; 19.1335x over previous
import functools

import jax
import jax.numpy as jnp
from jax import lax
from jax.experimental import pallas as pl
from jax.experimental.pallas import tpu as pltpu
from jax.experimental.pallas import tpu_sc as plsc

NC = 2
NS = 16
NW = NC * NS
CH = 128
BLK = 1024


def _hist_call(npad, ep, rpt, nch):
    mesh = plsc.VectorSubcoreMesh(core_axis_name="c", subcore_axis_name="s")

    @functools.partial(
        pl.kernel,
        mesh=mesh,
        out_type=jax.ShapeDtypeStruct((NC * npad,), jnp.float32),
        scratch_types=[
            pltpu.VMEM((CH,), jnp.int32),
            pltpu.VMEM((CH,), jnp.float32),
            pltpu.VMEM((rpt,), jnp.float32),
            pltpu.VMEM_SHARED((npad,), jnp.float32),
        ],
    )
    def hist(row_hbm, ones_hbm, zeros_hbm, out_hbm, idx_v, ones_v, buf_v, acc_sh):
        cid = lax.axis_index("c")
        sid = lax.axis_index("s")
        wid = sid * NC + cid
        pltpu.sync_copy(ones_hbm, ones_v)
        pltpu.sync_copy(zeros_hbm, buf_v)
        pltpu.sync_copy(buf_v, acc_sh.at[pl.ds(sid * rpt, rpt)])
        plsc.subcore_barrier()
        base = wid * (ep // NW)

        def body(i, carry):
            off = base + i * CH
            pltpu.sync_copy(row_hbm.at[pl.ds(off, CH)], idx_v)
            pltpu.sync_copy(ones_v, acc_sh.at[idx_v], add=True)
            return carry

        lax.fori_loop(0, nch, body, 0)
        plsc.subcore_barrier()
        pltpu.sync_copy(acc_sh.at[pl.ds(sid * rpt, rpt)], buf_v)
        pltpu.sync_copy(buf_v, out_hbm.at[pl.ds(cid * npad + sid * rpt, rpt)])

    return hist


def _agg_call(npad, d, ep, rpt, nch):
    mesh = plsc.VectorSubcoreMesh(core_axis_name="c", subcore_axis_name="s")

    @functools.partial(
        pl.kernel,
        mesh=mesh,
        out_type=jax.ShapeDtypeStruct((NC * npad, d), jnp.float32),
        scratch_types=[
            pltpu.VMEM((2, CH), jnp.int32),
            pltpu.VMEM((2, CH), jnp.int32),
            pltpu.VMEM((CH, d), jnp.float32),
            pltpu.VMEM((CH, d), jnp.float32),
            pltpu.VMEM_SHARED((npad, d), jnp.float32),
            pltpu.SemaphoreType.DMA,
            pltpu.SemaphoreType.DMA,
        ],
    )
    def agg(col_hbm, row_hbm, y_hbm, zeros_hbm, out_hbm,
            colv, rowv, rows0, rows1, acc_sh, sem0, sem1):
        cid = lax.axis_index("c")
        sid = lax.axis_index("s")
        wid = sid * NC + cid
        pltpu.sync_copy(zeros_hbm, rows0)
        for k in range(rpt // CH):
            pltpu.sync_copy(rows0, acc_sh.at[pl.ds(sid * rpt + k * CH, CH)])
        plsc.subcore_barrier()

        base = wid * (ep // NW)
        rows = (rows0, rows1)
        sems = (sem0, sem1)

        pltpu.sync_copy(col_hbm.at[pl.ds(base, CH)], colv.at[0])
        pltpu.sync_copy(row_hbm.at[pl.ds(base, CH)], rowv.at[0])
        pltpu.async_copy(y_hbm.at[colv.at[0]], rows0, sem0)

        def body(i, carry):
            cur = lax.rem(i, 2)
            nxt = lax.rem(i + 1, 2)

            @pl.when(i + 1 < nch)
            def _prefetch():
                off = base + (i + 1) * CH
                for b in range(2):
                    @pl.when(nxt == b)
                    def _():
                        pltpu.sync_copy(col_hbm.at[pl.ds(off, CH)], colv.at[b])
                        pltpu.sync_copy(row_hbm.at[pl.ds(off, CH)], rowv.at[b])
                        pltpu.async_copy(y_hbm.at[colv.at[b]], rows[b], sems[b])

            for b in range(2):
                @pl.when(cur == b)
                def _():
                    pltpu.make_async_copy(y_hbm.at[colv.at[b]], rows[b], sems[b]).wait()
                    pltpu.sync_copy(rows[b], acc_sh.at[rowv.at[b]], add=True)
            return carry

        lax.fori_loop(0, nch, body, 0)
        plsc.subcore_barrier()
        for k in range(rpt // CH):
            pltpu.sync_copy(acc_sh.at[pl.ds(sid * rpt + k * CH, CH)], rows0)
            pltpu.sync_copy(
                rows0, out_hbm.at[pl.ds(cid * npad + sid * rpt + k * CH, CH)])

    return agg


def _transform_kernel(d_ref, x_ref, w_ref, y_ref):
    deg = d_ref[0, :] + d_ref[1, :] + 2.0
    dinv = lax.rsqrt(deg)
    xw = jnp.dot(x_ref[...], w_ref[...], preferred_element_type=jnp.float32)
    y_ref[...] = dinv[:, None] * xw


def _final_kernel(d_ref, a_ref, y_ref, o_ref):
    deg = d_ref[0, :] + d_ref[1, :] + 2.0
    dinv = lax.rsqrt(deg)
    s = a_ref[0] + a_ref[1] + 2.0 * y_ref[...]
    o_ref[...] = jnp.maximum(dinv[:, None] * s, 0.0)


def kernel(x, edge_index, W):
    n, d_in = x.shape
    d_out = W.shape[1]
    e = edge_index.shape[1]

    npad = -(-n // (NS * CH)) * (NS * CH)
    ep = -(-e // (NW * CH)) * (NW * CH)
    rpt = npad // NS
    nch = ep // (NW * CH)

    row = edge_index[0].astype(jnp.int32)
    col = edge_index[1].astype(jnp.int32)
    pad_idx = jnp.full((ep - e,), npad - 1, dtype=jnp.int32)
    rowp = jnp.concatenate([row, pad_idx])
    colp = jnp.concatenate([col, pad_idx])
    xp = jnp.pad(x, ((0, npad - n), (0, 0)))

    ones_ch = jnp.ones((CH,), jnp.float32)
    zeros_rpt = jnp.zeros((rpt,), jnp.float32)
    zeros_blk = jnp.zeros((CH, d_out), jnp.float32)

    degp = _hist_call(npad, ep, rpt, nch)(rowp, ones_ch, zeros_rpt)
    deg2 = degp.reshape(NC, npad)

    nb = npad // BLK
    y = pl.pallas_call(
        _transform_kernel,
        grid=(nb,),
        in_specs=[
            pl.BlockSpec((NC, BLK), lambda i: (0, i)),
            pl.BlockSpec((BLK, d_in), lambda i: (i, 0)),
            pl.BlockSpec((d_in, d_out), lambda i: (0, 0)),
        ],
        out_specs=pl.BlockSpec((BLK, d_out), lambda i: (i, 0)),
        out_shape=jax.ShapeDtypeStruct((npad, d_out), jnp.float32),
    )(deg2, xp, W)

    aggp = _agg_call(npad, d_out, ep, rpt, nch)(colp, rowp, y, zeros_blk)
    agg3 = aggp.reshape(NC, npad, d_out)

    out = pl.pallas_call(
        _final_kernel,
        grid=(nb,),
        in_specs=[
            pl.BlockSpec((NC, BLK), lambda i: (0, i)),
            pl.BlockSpec((NC, BLK, d_out), lambda i: (0, i, 0)),
            pl.BlockSpec((BLK, d_out), lambda i: (i, 0)),
        ],
        out_specs=pl.BlockSpec((BLK, d_out), lambda i: (i, 0)),
        out_shape=jax.ShapeDtypeStruct((npad, d_out), jnp.float32),
    )(deg2, agg3, y)

    return out[:n]


# speedup vs baseline: 25.2864x; 1.3216x over previous
import functools
import math

import jax
import jax.numpy as jnp
from jax import lax
from jax.experimental import pallas as pl
from jax.experimental.pallas import tpu as pltpu
from jax.experimental.pallas import tpu_sc as plsc

NC = 2
NS = 16
NW = NC * NS
CH = 120
BLK = 1024


def _hist_call(npad, ep, rpt, nch):
    mesh = plsc.VectorSubcoreMesh(core_axis_name="c", subcore_axis_name="s")

    @functools.partial(
        pl.kernel,
        mesh=mesh,
        out_type=jax.ShapeDtypeStruct((NC * npad,), jnp.float32),
        scratch_types=[
            pltpu.VMEM((4, CH), jnp.int32),
            pltpu.VMEM((CH,), jnp.float32),
            pltpu.VMEM((rpt,), jnp.float32),
            pltpu.VMEM_SHARED((npad,), jnp.float32),
            [pltpu.SemaphoreType.DMA] * 4,
        ],
    )
    def hist(row_hbm, ones_hbm, zeros_hbm, out_hbm,
             ibuf, ones_v, buf_v, acc_sh, sems):
        cid = lax.axis_index("c")
        sid = lax.axis_index("s")
        wid = sid * NC + cid
        pltpu.sync_copy(ones_hbm, ones_v)
        pltpu.sync_copy(zeros_hbm, buf_v)
        pltpu.sync_copy(buf_v, acc_sh.at[pl.ds(sid * rpt, rpt)])
        plsc.subcore_barrier()
        base = wid * (ep // NW)

        pltpu.sync_copy(row_hbm.at[pl.ds(base, CH)], ibuf.at[0])
        pltpu.sync_copy(row_hbm.at[pl.ds(base + CH, CH)], ibuf.at[1])

        def group(g, carry):
            for b in range(4):
                i = g * 4 + b
                bf = (b + 2) % 4

                @pl.when(i + 2 < nch)
                def _fetch():
                    off = base + (i + 2) * CH
                    pltpu.async_copy(
                        row_hbm.at[pl.ds(off, CH)], ibuf.at[bf], sems[bf])

                @pl.when(i < nch)
                def _scat():
                    if b in (0, 1):
                        @pl.when(g > 0)
                        def _wait():
                            pltpu.make_async_copy(
                                row_hbm.at[pl.ds(base, CH)], ibuf.at[b],
                                sems[b]).wait()
                    else:
                        pltpu.make_async_copy(
                            row_hbm.at[pl.ds(base, CH)], ibuf.at[b],
                            sems[b]).wait()
                    pltpu.sync_copy(ones_v, acc_sh.at[ibuf.at[b]], add=True)
            return carry

        lax.fori_loop(0, (nch + 3) // 4, group, 0)
        plsc.subcore_barrier()
        pltpu.sync_copy(acc_sh.at[pl.ds(sid * rpt, rpt)], buf_v)
        pltpu.sync_copy(buf_v, out_hbm.at[pl.ds(cid * npad + sid * rpt, rpt)])

    return hist


def _agg_call(npad, d, ep, rpt, nch):
    mesh = plsc.VectorSubcoreMesh(core_axis_name="c", subcore_axis_name="s")
    rc = math.gcd(rpt, CH)

    @functools.partial(
        pl.kernel,
        mesh=mesh,
        out_type=jax.ShapeDtypeStruct((NC * npad, d), jnp.float32),
        scratch_types=[
            pltpu.VMEM((3, CH), jnp.int32),
            pltpu.VMEM((3, CH), jnp.int32),
            pltpu.VMEM((3, CH, d), jnp.float32),
            pltpu.VMEM_SHARED((npad, d), jnp.float32),
            [pltpu.SemaphoreType.DMA] * 3,
            [pltpu.SemaphoreType.DMA] * 3,
        ],
    )
    def agg(col_hbm, row_hbm, y_hbm, zeros_hbm, out_hbm,
            cbuf, rbuf, rows, acc_sh, sis, sgs):
        cid = lax.axis_index("c")
        sid = lax.axis_index("s")
        wid = sid * NC + cid
        pltpu.sync_copy(zeros_hbm, rows.at[0, pl.ds(0, rc)])
        for k in range(rpt // rc):
            pltpu.sync_copy(
                rows.at[0, pl.ds(0, rc)],
                acc_sh.at[pl.ds(sid * rpt + k * rc, rc)])
        plsc.subcore_barrier()

        base = wid * (ep // NW)

        for j in range(2):
            pltpu.sync_copy(col_hbm.at[pl.ds(base + j * CH, CH)], cbuf.at[j])
            pltpu.sync_copy(row_hbm.at[pl.ds(base + j * CH, CH)], rbuf.at[j])
        pltpu.async_copy(y_hbm.at[cbuf.at[0]], rows.at[0], sgs[0])

        def group(g, carry):
            for b in range(3):
                i = g * 3 + b
                b2 = (b + 2) % 3
                b1 = (b + 1) % 3

                @pl.when(i + 2 < nch)
                def _fetch():
                    off = base + (i + 2) * CH
                    pltpu.async_copy(
                        col_hbm.at[pl.ds(off, CH)], cbuf.at[b2], sis[b2])
                    pltpu.async_copy(
                        row_hbm.at[pl.ds(off, CH)], rbuf.at[b2], sis[b2])

                @pl.when(i + 1 < nch)
                def _gather():
                    if b == 0:
                        @pl.when(g > 0)
                        def _():
                            pltpu.make_async_copy(
                                col_hbm.at[pl.ds(base, CH)], cbuf.at[b1],
                                sis[b1]).wait()
                            pltpu.make_async_copy(
                                row_hbm.at[pl.ds(base, CH)], rbuf.at[b1],
                                sis[b1]).wait()
                    else:
                        pltpu.make_async_copy(
                            col_hbm.at[pl.ds(base, CH)], cbuf.at[b1],
                            sis[b1]).wait()
                        pltpu.make_async_copy(
                            row_hbm.at[pl.ds(base, CH)], rbuf.at[b1],
                            sis[b1]).wait()
                    pltpu.async_copy(
                        y_hbm.at[cbuf.at[b1]], rows.at[b1], sgs[b1])

                @pl.when(i < nch)
                def _scat():
                    pltpu.make_async_copy(
                        y_hbm.at[cbuf.at[b]], rows.at[b], sgs[b]).wait()
                    pltpu.sync_copy(
                        rows.at[b], acc_sh.at[rbuf.at[b]], add=True)
            return carry

        lax.fori_loop(0, (nch + 2) // 3, group, 0)
        plsc.subcore_barrier()
        for k in range(rpt // rc):
            pltpu.sync_copy(
                acc_sh.at[pl.ds(sid * rpt + k * rc, rc)],
                rows.at[0, pl.ds(0, rc)])
            pltpu.sync_copy(
                rows.at[0, pl.ds(0, rc)],
                out_hbm.at[pl.ds(cid * npad + sid * rpt + k * rc, rc)])

    return agg


def _transform_kernel(d_ref, x_ref, w_ref, y_ref):
    deg = d_ref[0, :] + d_ref[1, :] + 2.0
    dinv = lax.rsqrt(deg)
    xw = jnp.dot(x_ref[...], w_ref[...], preferred_element_type=jnp.float32)
    y_ref[...] = dinv[:, None] * xw


def _final_kernel(d_ref, a_ref, y_ref, o_ref):
    deg = d_ref[0, :] + d_ref[1, :] + 2.0
    dinv = lax.rsqrt(deg)
    s = a_ref[0] + a_ref[1] + 2.0 * y_ref[...]
    o_ref[...] = jnp.maximum(dinv[:, None] * s, 0.0)


def kernel(x, edge_index, W):
    n, d_in = x.shape
    d_out = W.shape[1]
    e = edge_index.shape[1]

    npad = -(-n // (NS * 128)) * (NS * 128)
    ep = -(-e // (NW * CH)) * (NW * CH)
    rpt = npad // NS
    nch = ep // (NW * CH)

    row = edge_index[0].astype(jnp.int32)
    col = edge_index[1].astype(jnp.int32)
    pad_idx = jnp.full((ep - e,), npad - 1, dtype=jnp.int32)
    rowp = jnp.concatenate([row, pad_idx])
    colp = jnp.concatenate([col, pad_idx])
    xp = jnp.pad(x, ((0, npad - n), (0, 0)))

    rc = math.gcd(rpt, CH)
    ones_ch = jnp.ones((CH,), jnp.float32)
    zeros_rpt = jnp.zeros((rpt,), jnp.float32)
    zeros_blk = jnp.zeros((rc, d_out), jnp.float32)

    degp = _hist_call(npad, ep, rpt, nch)(rowp, ones_ch, zeros_rpt)
    deg2 = degp.reshape(NC, npad)

    nb = npad // BLK
    y = pl.pallas_call(
        _transform_kernel,
        grid=(nb,),
        in_specs=[
            pl.BlockSpec((NC, BLK), lambda i: (0, i)),
            pl.BlockSpec((BLK, d_in), lambda i: (i, 0)),
            pl.BlockSpec((d_in, d_out), lambda i: (0, 0)),
        ],
        out_specs=pl.BlockSpec((BLK, d_out), lambda i: (i, 0)),
        out_shape=jax.ShapeDtypeStruct((npad, d_out), jnp.float32),
    )(deg2, xp, W)

    aggp = _agg_call(npad, d_out, ep, rpt, nch)(colp, rowp, y, zeros_blk)
    agg3 = aggp.reshape(NC, npad, d_out)

    out = pl.pallas_call(
        _final_kernel,
        grid=(nb,),
        in_specs=[
            pl.BlockSpec((NC, BLK), lambda i: (0, i)),
            pl.BlockSpec((NC, BLK, d_out), lambda i: (0, i, 0)),
            pl.BlockSpec((BLK, d_out), lambda i: (i, 0)),
        ],
        out_specs=pl.BlockSpec((BLK, d_out), lambda i: (i, 0)),
        out_shape=jax.ShapeDtypeStruct((npad, d_out), jnp.float32),
    )(deg2, agg3, y)

    return out[:n]


# speedup vs baseline: 39.3328x; 1.5555x over previous
import functools
import math

import jax
import jax.numpy as jnp
from jax import lax
from jax.experimental import pallas as pl
from jax.experimental.pallas import tpu as pltpu
from jax.experimental.pallas import tpu_sc as plsc

NC = 2
NS = 16
NW = NC * NS
CH = 80
NR = 4
BLK = 1024


def _hist_call(npad, ep, rpt, nch):
    mesh = plsc.VectorSubcoreMesh(core_axis_name="c", subcore_axis_name="s")

    @functools.partial(
        pl.kernel,
        mesh=mesh,
        out_type=jax.ShapeDtypeStruct((NC * npad,), jnp.float32),
        scratch_types=[
            pltpu.VMEM((NR, CH), jnp.int32),
            pltpu.VMEM((CH,), jnp.float32),
            pltpu.VMEM((rpt,), jnp.float32),
            pltpu.VMEM_SHARED((npad,), jnp.float32),
            [pltpu.SemaphoreType.DMA] * NR,
        ],
    )
    def hist(row_hbm, ones_hbm, zeros_hbm, out_hbm,
             ibuf, ones_v, buf_v, acc_sh, sems):
        cid = lax.axis_index("c")
        sid = lax.axis_index("s")
        wid = sid * NC + cid
        pltpu.sync_copy(ones_hbm, ones_v)
        pltpu.sync_copy(zeros_hbm, buf_v)
        pltpu.sync_copy(buf_v, acc_sh.at[pl.ds(sid * rpt, rpt)])
        plsc.subcore_barrier()
        base = wid * (ep // NW)

        pltpu.sync_copy(row_hbm.at[pl.ds(base, CH)], ibuf.at[0])
        pltpu.sync_copy(row_hbm.at[pl.ds(base + CH, CH)], ibuf.at[1])

        def group(g, carry):
            for b in range(NR):
                i = g * NR + b
                bf = (b + 2) % NR

                @pl.when(i + 2 < nch)
                def _fetch():
                    off = base + (i + 2) * CH
                    pltpu.async_copy(
                        row_hbm.at[pl.ds(off, CH)], ibuf.at[bf], sems[bf])

                @pl.when(i < nch)
                def _scat():
                    if b in (0, 1):
                        @pl.when(g > 0)
                        def _wait():
                            pltpu.make_async_copy(
                                row_hbm.at[pl.ds(base, CH)], ibuf.at[b],
                                sems[b]).wait()
                    else:
                        pltpu.make_async_copy(
                            row_hbm.at[pl.ds(base, CH)], ibuf.at[b],
                            sems[b]).wait()
                    pltpu.sync_copy(ones_v, acc_sh.at[ibuf.at[b]], add=True)
            return carry

        lax.fori_loop(0, (nch + NR - 1) // NR, group, 0)
        plsc.subcore_barrier()
        pltpu.sync_copy(acc_sh.at[pl.ds(sid * rpt, rpt)], buf_v)
        pltpu.sync_copy(buf_v, out_hbm.at[pl.ds(cid * npad + sid * rpt, rpt)])

    return hist


def _agg_call(npad, d, ep, rpt, nch):
    mesh = plsc.VectorSubcoreMesh(core_axis_name="c", subcore_axis_name="s")
    rc = math.gcd(rpt, CH)

    @functools.partial(
        pl.kernel,
        mesh=mesh,
        out_type=jax.ShapeDtypeStruct((NC * npad, d), jnp.float32),
        scratch_types=[
            pltpu.VMEM((NR, CH), jnp.int32),
            pltpu.VMEM((NR, CH), jnp.int32),
            pltpu.VMEM((NR, CH, d), jnp.float32),
            pltpu.VMEM_SHARED((npad, d), jnp.float32),
            [pltpu.SemaphoreType.DMA] * NR,
            [pltpu.SemaphoreType.DMA] * NR,
        ],
    )
    def agg(col_hbm, row_hbm, y_hbm, zeros_hbm, out_hbm,
            cbuf, rbuf, rows, acc_sh, sis, sgs):
        cid = lax.axis_index("c")
        sid = lax.axis_index("s")
        wid = sid * NC + cid
        pltpu.sync_copy(zeros_hbm, rows.at[0, pl.ds(0, rc)])
        for k in range(rpt // rc):
            pltpu.sync_copy(
                rows.at[0, pl.ds(0, rc)],
                acc_sh.at[pl.ds(sid * rpt + k * rc, rc)])
        plsc.subcore_barrier()

        base = wid * (ep // NW)

        for j in range(2):
            pltpu.sync_copy(col_hbm.at[pl.ds(base + j * CH, CH)], cbuf.at[j])
            pltpu.sync_copy(row_hbm.at[pl.ds(base + j * CH, CH)], rbuf.at[j])
        pltpu.async_copy(y_hbm.at[cbuf.at[0]], rows.at[0], sgs[0])

        def group(g, carry):
            for b in range(NR):
                i = g * NR + b
                b2 = (b + 2) % NR
                b1 = (b + 1) % NR

                @pl.when(i + 2 < nch)
                def _fetch():
                    off = base + (i + 2) * CH
                    pltpu.async_copy(
                        col_hbm.at[pl.ds(off, CH)], cbuf.at[b2], sis[b2])
                    pltpu.async_copy(
                        row_hbm.at[pl.ds(off, CH)], rbuf.at[b2], sis[b2])

                @pl.when(i + 1 < nch)
                def _gather():
                    if b == 0:
                        @pl.when(g > 0)
                        def _():
                            pltpu.make_async_copy(
                                col_hbm.at[pl.ds(base, CH)], cbuf.at[b1],
                                sis[b1]).wait()
                            pltpu.make_async_copy(
                                row_hbm.at[pl.ds(base, CH)], rbuf.at[b1],
                                sis[b1]).wait()
                    else:
                        pltpu.make_async_copy(
                            col_hbm.at[pl.ds(base, CH)], cbuf.at[b1],
                            sis[b1]).wait()
                        pltpu.make_async_copy(
                            row_hbm.at[pl.ds(base, CH)], rbuf.at[b1],
                            sis[b1]).wait()
                    pltpu.async_copy(
                        y_hbm.at[cbuf.at[b1]], rows.at[b1], sgs[b1])

                @pl.when(i < nch)
                def _scat():
                    pltpu.make_async_copy(
                        y_hbm.at[cbuf.at[b]], rows.at[b], sgs[b]).wait()
                    pltpu.sync_copy(
                        rows.at[b], acc_sh.at[rbuf.at[b]], add=True)
            return carry

        lax.fori_loop(0, (nch + NR - 1) // NR, group, 0)
        plsc.subcore_barrier()
        for k in range(rpt // rc):
            pltpu.sync_copy(
                acc_sh.at[pl.ds(sid * rpt + k * rc, rc)],
                rows.at[0, pl.ds(0, rc)])
            pltpu.sync_copy(
                rows.at[0, pl.ds(0, rc)],
                out_hbm.at[pl.ds(cid * npad + sid * rpt + k * rc, rc)])

    return agg


def _transform_kernel(d_ref, x_ref, w_ref, y_ref):
    deg = d_ref[0, :] + d_ref[1, :] + 2.0
    dinv = lax.rsqrt(deg)
    xw = jnp.dot(x_ref[...], w_ref[...], preferred_element_type=jnp.float32)
    y_ref[...] = dinv[:, None] * xw


def _final_kernel(d_ref, a_ref, y_ref, o_ref):
    deg = d_ref[0, :] + d_ref[1, :] + 2.0
    dinv = lax.rsqrt(deg)
    s = a_ref[0] + a_ref[1] + 2.0 * y_ref[...]
    o_ref[...] = jnp.maximum(dinv[:, None] * s, 0.0)


def kernel(x, edge_index, W):
    n, d_in = x.shape
    d_out = W.shape[1]
    e = edge_index.shape[1]

    npad = -(-n // (NS * CH)) * (NS * CH)
    ep = -(-e // (NW * CH)) * (NW * CH)
    rpt = npad // NS
    nch = ep // (NW * CH)

    row = edge_index[0].astype(jnp.int32)
    col = edge_index[1].astype(jnp.int32)
    if ep != e:
        pad_idx = jnp.full((ep - e,), npad - 1, dtype=jnp.int32)
        row = jnp.concatenate([row, pad_idx])
        col = jnp.concatenate([col, pad_idx])

    rc = math.gcd(rpt, CH)
    ones_ch = jnp.ones((CH,), jnp.float32)
    zeros_rpt = jnp.zeros((rpt,), jnp.float32)
    zeros_blk = jnp.zeros((rc, d_out), jnp.float32)

    degp = _hist_call(npad, ep, rpt, nch)(row, ones_ch, zeros_rpt)
    deg2 = degp.reshape(NC, npad)

    nb = -(-n // BLK)
    y = pl.pallas_call(
        _transform_kernel,
        grid=(nb,),
        in_specs=[
            pl.BlockSpec((NC, BLK), lambda i: (0, i)),
            pl.BlockSpec((BLK, d_in), lambda i: (i, 0)),
            pl.BlockSpec((d_in, d_out), lambda i: (0, 0)),
        ],
        out_specs=pl.BlockSpec((BLK, d_out), lambda i: (i, 0)),
        out_shape=jax.ShapeDtypeStruct((n, d_out), jnp.float32),
    )(deg2, x, W)

    aggp = _agg_call(npad, d_out, ep, rpt, nch)(col, row, y, zeros_blk)
    agg3 = aggp.reshape(NC, npad, d_out)

    out = pl.pallas_call(
        _final_kernel,
        grid=(nb,),
        in_specs=[
            pl.BlockSpec((NC, BLK), lambda i: (0, i)),
            pl.BlockSpec((NC, BLK, d_out), lambda i: (0, i, 0)),
            pl.BlockSpec((BLK, d_out), lambda i: (i, 0)),
        ],
        out_specs=pl.BlockSpec((BLK, d_out), lambda i: (i, 0)),
        out_shape=jax.ShapeDtypeStruct((n, d_out), jnp.float32),
    )(deg2, agg3, y)

    return out


# speedup vs baseline: 47.7177x; 1.2132x over previous
import functools
import math

import jax
import jax.numpy as jnp
from jax import lax
from jax.experimental import pallas as pl
from jax.experimental.pallas import tpu as pltpu
from jax.experimental.pallas import tpu_sc as plsc

NC = 2
NS = 16
NW = NC * NS
CH = 80
NR = 4
GB = 5
BLK = 1024


def _hist_call(npad, ep, rpt, nch):
    mesh = plsc.VectorSubcoreMesh(core_axis_name="c", subcore_axis_name="s")
    ng = nch // GB

    @functools.partial(
        pl.kernel,
        mesh=mesh,
        out_type=jax.ShapeDtypeStruct((NC * npad,), jnp.float32),
        scratch_types=[
            pltpu.VMEM((2, GB, CH), jnp.int32),
            pltpu.VMEM((CH,), jnp.float32),
            pltpu.VMEM((rpt,), jnp.float32),
            pltpu.VMEM_SHARED((npad,), jnp.float32),
            [pltpu.SemaphoreType.DMA] * 2,
        ],
    )
    def hist(rowg_hbm, ones_hbm, zeros_hbm, out_hbm,
             ibuf, ones_v, buf_v, acc_sh, sems):
        cid = lax.axis_index("c")
        sid = lax.axis_index("s")
        wid = sid * NC + cid
        pltpu.sync_copy(ones_hbm, ones_v)
        pltpu.sync_copy(zeros_hbm, buf_v)
        pltpu.sync_copy(buf_v, acc_sh.at[pl.ds(sid * rpt, rpt)])
        plsc.subcore_barrier()
        gbase = wid * ng

        pltpu.sync_copy(rowg_hbm.at[gbase], ibuf.at[0])

        def group(g, carry):
            for b in range(2):
                j = g * 2 + b
                nb = 1 - b

                @pl.when(j + 1 < ng)
                def _fetch():
                    pltpu.async_copy(
                        rowg_hbm.at[gbase + j + 1], ibuf.at[nb], sems[nb])

                @pl.when(j < ng)
                def _scat():
                    if b == 0:
                        @pl.when(g > 0)
                        def _wait():
                            pltpu.make_async_copy(
                                rowg_hbm.at[gbase], ibuf.at[b], sems[b]).wait()
                    else:
                        pltpu.make_async_copy(
                            rowg_hbm.at[gbase], ibuf.at[b], sems[b]).wait()
                    for k in range(GB):
                        pltpu.sync_copy(
                            ones_v, acc_sh.at[ibuf.at[b, k]], add=True)
            return carry

        lax.fori_loop(0, (ng + 1) // 2, group, 0)
        plsc.subcore_barrier()
        pltpu.sync_copy(acc_sh.at[pl.ds(sid * rpt, rpt)], buf_v)
        pltpu.sync_copy(buf_v, out_hbm.at[pl.ds(cid * npad + sid * rpt, rpt)])

    return hist


def _agg_call(npad, d, ep, rpt, nch):
    mesh = plsc.VectorSubcoreMesh(core_axis_name="c", subcore_axis_name="s")
    rc = math.gcd(rpt, CH)

    @functools.partial(
        pl.kernel,
        mesh=mesh,
        out_type=jax.ShapeDtypeStruct((NC * npad, d), jnp.float32),
        scratch_types=[
            pltpu.VMEM((NR, CH), jnp.int32),
            pltpu.VMEM((NR, CH), jnp.int32),
            pltpu.VMEM((NR, CH, d), jnp.float32),
            pltpu.VMEM_SHARED((npad, d), jnp.float32),
            [pltpu.SemaphoreType.DMA] * NR,
            [pltpu.SemaphoreType.DMA] * NR,
            [pltpu.SemaphoreType.DMA] * NR,
        ],
    )
    def agg(ef_hbm, y_hbm, zeros_hbm, out_hbm,
            cbuf, rbuf, rows, acc_sh, sis, sgs, sss):
        cid = lax.axis_index("c")
        sid = lax.axis_index("s")
        wid = sid * NC + cid
        pltpu.sync_copy(zeros_hbm, rows.at[0, pl.ds(0, rc)])
        for k in range(rpt // rc):
            pltpu.sync_copy(
                rows.at[0, pl.ds(0, rc)],
                acc_sh.at[pl.ds(sid * rpt + k * rc, rc)])
        plsc.subcore_barrier()

        base = wid * (ep // NW)
        cbase = ep + base

        for j in range(2):
            pltpu.sync_copy(ef_hbm.at[pl.ds(cbase + j * CH, CH)], cbuf.at[j])
            pltpu.sync_copy(ef_hbm.at[pl.ds(base + j * CH, CH)], rbuf.at[j])
        pltpu.async_copy(y_hbm.at[cbuf.at[0]], rows.at[0], sgs[0])

        def group(g, carry):
            for b in range(NR):
                i = g * NR + b
                b2 = (b + 2) % NR
                b1 = (b + 1) % NR

                @pl.when(i + 2 < nch)
                def _fetch():
                    if b in (0, 1):
                        @pl.when(g > 0)
                        def _ws():
                            pltpu.make_async_copy(
                                rows.at[b2], acc_sh.at[rbuf.at[b2]],
                                sss[b2]).wait()
                    else:
                        pltpu.make_async_copy(
                            rows.at[b2], acc_sh.at[rbuf.at[b2]],
                            sss[b2]).wait()
                    off = base + (i + 2) * CH
                    pltpu.async_copy(
                        ef_hbm.at[pl.ds(ep + off, CH)], cbuf.at[b2], sis[b2])
                    pltpu.async_copy(
                        ef_hbm.at[pl.ds(off, CH)], rbuf.at[b2], sis[b2])

                @pl.when(i + 1 < nch)
                def _gather():
                    if b == 0:
                        @pl.when(g > 0)
                        def _wi():
                            pltpu.make_async_copy(
                                ef_hbm.at[pl.ds(cbase, CH)], cbuf.at[b1],
                                sis[b1]).wait()
                            pltpu.make_async_copy(
                                ef_hbm.at[pl.ds(base, CH)], rbuf.at[b1],
                                sis[b1]).wait()
                    else:
                        pltpu.make_async_copy(
                            ef_hbm.at[pl.ds(cbase, CH)], cbuf.at[b1],
                            sis[b1]).wait()
                        pltpu.make_async_copy(
                            ef_hbm.at[pl.ds(base, CH)], rbuf.at[b1],
                            sis[b1]).wait()
                    pltpu.async_copy(
                        y_hbm.at[cbuf.at[b1]], rows.at[b1], sgs[b1])

                @pl.when(i < nch)
                def _scat():
                    pltpu.make_async_copy(
                        y_hbm.at[cbuf.at[b]], rows.at[b], sgs[b]).wait()
                    pltpu.async_copy(
                        rows.at[b], acc_sh.at[rbuf.at[b]], sss[b], add=True)
            return carry

        lax.fori_loop(0, (nch + NR - 1) // NR, group, 0)
        for b in range(NR):
            pltpu.make_async_copy(
                rows.at[b], acc_sh.at[rbuf.at[b]], sss[b]).wait()
        plsc.subcore_barrier()
        for k in range(rpt // rc):
            pltpu.sync_copy(
                acc_sh.at[pl.ds(sid * rpt + k * rc, rc)],
                rows.at[0, pl.ds(0, rc)])
            pltpu.sync_copy(
                rows.at[0, pl.ds(0, rc)],
                out_hbm.at[pl.ds(cid * npad + sid * rpt + k * rc, rc)])

    return agg


def _transform_kernel(d_ref, x_ref, w_ref, y_ref):
    deg = d_ref[0, :] + d_ref[1, :] + 2.0
    dinv = lax.rsqrt(deg)
    xw = jnp.dot(x_ref[...], w_ref[...], preferred_element_type=jnp.float32)
    y_ref[...] = dinv[:, None] * xw


def _final_kernel(d_ref, a_ref, y_ref, o_ref):
    deg = d_ref[0, :] + d_ref[1, :] + 2.0
    dinv = lax.rsqrt(deg)
    s = a_ref[0] + a_ref[1] + 2.0 * y_ref[...]
    o_ref[...] = jnp.maximum(dinv[:, None] * s, 0.0)


def kernel(x, edge_index, W):
    n, d_in = x.shape
    d_out = W.shape[1]
    e = edge_index.shape[1]

    npad = -(-n // (NS * CH)) * (NS * CH)
    ep = -(-e // (NW * CH)) * (NW * CH)
    rpt = npad // NS
    nch = ep // (NW * CH)

    ei = edge_index.astype(jnp.int32)
    if ep != e:
        pad_idx = jnp.full((2, ep - e), npad - 1, dtype=jnp.int32)
        ei = jnp.concatenate([ei, pad_idx], axis=1)
    eflat = ei.reshape(-1)
    rowg = eflat.reshape(-1, GB, CH)

    rc = math.gcd(rpt, CH)
    ones_ch = jnp.ones((CH,), jnp.float32)
    zeros_rpt = jnp.zeros((rpt,), jnp.float32)
    zeros_blk = jnp.zeros((rc, d_out), jnp.float32)

    degp = _hist_call(npad, ep, rpt, nch)(rowg, ones_ch, zeros_rpt)
    deg2 = degp.reshape(NC, npad)

    nb = -(-n // BLK)
    y = pl.pallas_call(
        _transform_kernel,
        grid=(nb,),
        in_specs=[
            pl.BlockSpec((NC, BLK), lambda i: (0, i)),
            pl.BlockSpec((BLK, d_in), lambda i: (i, 0)),
            pl.BlockSpec((d_in, d_out), lambda i: (0, 0)),
        ],
        out_specs=pl.BlockSpec((BLK, d_out), lambda i: (i, 0)),
        out_shape=jax.ShapeDtypeStruct((n, d_out), jnp.float32),
    )(deg2, x, W)

    aggp = _agg_call(npad, d_out, ep, rpt, nch)(eflat, y, zeros_blk)
    agg3 = aggp.reshape(NC, npad, d_out)

    out = pl.pallas_call(
        _final_kernel,
        grid=(nb,),
        in_specs=[
            pl.BlockSpec((NC, BLK), lambda i: (0, i)),
            pl.BlockSpec((NC, BLK, d_out), lambda i: (0, i, 0)),
            pl.BlockSpec((BLK, d_out), lambda i: (i, 0)),
        ],
        out_specs=pl.BlockSpec((BLK, d_out), lambda i: (i, 0)),
        out_shape=jax.ShapeDtypeStruct((n, d_out), jnp.float32),
    )(deg2, agg3, y)

    return out
